# baseline (speedup 1.0000x reference)
import jax
import jax.numpy as jnp
from jax.experimental import pallas as pl
from jax.experimental.pallas import tpu as pltpu

_C = 1024


def _make_body(B, S, P, H, C, k_last):
    def body(p_ref, x_ref, out_ref, carry, psem):
        k = pl.program_id(1)

        @pl.when(k == 0)
        def _():
            cp = pltpu.make_async_copy(p_ref, carry, psem)
            cp.start()
            cp.wait()

        @pl.when(k < k_last)
        def _():
            out_ref[0, :P] = carry[...]
            out_ref[0, P:] = x_ref[0, : C - P]
            carry[...] = x_ref[0, C - P :]

        @pl.when(k == k_last)
        def _():
            out_ref[0, :P] = carry[...]

    return body


def kernel(input_embeddings, prompt_weight):
    B, S, H = input_embeddings.shape
    P = prompt_weight.shape[0]
    C = _C
    assert S % C == 0
    x_blocks = S // C
    k_last = x_blocks
    return pl.pallas_call(
        _make_body(B, S, P, H, C, k_last),
        grid=(B, k_last + 1),
        out_shape=jax.ShapeDtypeStruct((B, P + S, H), input_embeddings.dtype),
        in_specs=[
            pl.BlockSpec(memory_space=pltpu.MemorySpace.HBM),
            pl.BlockSpec((1, C, H), lambda b, k: (b, jnp.minimum(k, x_blocks - 1), 0)),
        ],
        out_specs=pl.BlockSpec((1, C, H), lambda b, k: (b, k, 0)),
        scratch_shapes=[
            pltpu.VMEM((P, H), input_embeddings.dtype),
            pltpu.SemaphoreType.DMA,
        ],
    )(prompt_weight, input_embeddings)


# speedup vs baseline: 1.0512x; 1.0512x over previous
import jax
import jax.numpy as jnp
from jax.experimental import pallas as pl
from jax.experimental.pallas import tpu as pltpu

_C = 1024


def _make_body(B, S, P, H, C, k_last):
    def body(p_ref, x_ref, out_ref, carry):
        k = pl.program_id(1)

        @pl.when(k == 0)
        def _():
            carry[...] = p_ref[...]

        @pl.when(k < k_last)
        def _():
            out_ref[0, :P] = carry[...]
            out_ref[0, P:] = x_ref[0, : C - P]
            carry[...] = x_ref[0, C - P :]

        @pl.when(k == k_last)
        def _():
            out_ref[0, :P] = carry[...]

    return body


def kernel(input_embeddings, prompt_weight):
    B, S, H = input_embeddings.shape
    P = prompt_weight.shape[0]
    C = _C
    assert S % C == 0
    x_blocks = S // C
    k_last = x_blocks
    return pl.pallas_call(
        _make_body(B, S, P, H, C, k_last),
        grid=(B, k_last + 1),
        out_shape=jax.ShapeDtypeStruct((B, P + S, H), input_embeddings.dtype),
        in_specs=[
            pl.BlockSpec((P, H), lambda b, k: (0, 0)),
            pl.BlockSpec((1, C, H), lambda b, k: (b, jnp.minimum(k, x_blocks - 1), 0)),
        ],
        out_specs=pl.BlockSpec((1, C, H), lambda b, k: (b, k, 0)),
        scratch_shapes=[pltpu.VMEM((P, H), input_embeddings.dtype)],
    )(prompt_weight, input_embeddings)
